# Initial kernel scaffold; baseline (speedup 1.0000x reference)
import jax
import jax.numpy as jnp
from jax.experimental import pallas as pl
from jax.experimental.pallas import tpu as pltpu
from jax.experimental.pallas import tpu_sc as plsc

N = 4096
D = 128
NHID = 128
NCLASS = 128
ALPHA = 0.2
R = 256
GW = 128
NEG = -9e15


def _sc_gather(table, idx):
    n = idx.shape[0]
    d = table.shape[1]
    idx2 = idx.reshape(1, n)
    mesh = plsc.VectorSubcoreMesh(core_axis_name="core",
                                  subcore_axis_name="subcore")

    @pl.kernel(out_type=jax.ShapeDtypeStruct((n, d), table.dtype), mesh=mesh)
    def gather_kernel(tab_hbm, i_hbm, o_hbm):
        def body(i_vmem, o_vmem):
            pltpu.sync_copy(tab_hbm.at[i_vmem.at[0]], o_vmem)

        pltpu.emit_pipeline(
            body,
            grid=(n // GW,),
            in_specs=[pl.BlockSpec((1, GW), index_map=lambda i: (0, i))],
            out_specs=[pl.BlockSpec((GW, d), index_map=lambda i: (i, 0))],
            core_axis_name=("core", "subcore"),
            dimension_semantics=(pltpu.PARALLEL,),
        )(i_hbm, o_hbm)

    return gather_kernel(table, idx2)


def _prewh_body(x_ref, w1a_ref, w1b_ref, wa_ref, wb_ref):
    x = x_ref[...]
    wa_ref[...] = jnp.dot(x, w1a_ref[...], preferred_element_type=jnp.float32)
    wb_ref[...] = jnp.dot(x, w1b_ref[...], preferred_element_type=jnp.float32)


def _masked_softmax(e, mask):
    e = jnp.where(e >= 0, e, ALPHA * e)
    e = jnp.where(mask, e, NEG)
    m = jnp.max(e, axis=1, keepdims=True)
    p = jnp.exp(e - m)
    return p / jnp.sum(p, axis=1, keepdims=True)


def _elu(x):
    return jnp.where(x > 0, x, jnp.expm1(x))


def _layer1_body(adj_ref, wa_ref, wb_ref, a1a_ref, a1b_ref, wout_ref,
                 who_ref):
    i = pl.program_id(0)
    mask = adj_ref[...] > 0.0

    def head(w_ref, a_ref):
        w = w_ref[...]
        a_row = a_ref[0, :]
        wrows = w_ref[pl.ds(i * R, R), :]
        f1 = jnp.sum(wrows * a_row[:D][None, :], axis=1)
        f2 = jnp.sum(w * a_row[D:][None, :], axis=1)
        p = _masked_softmax(f1[:, None] + f2[None, :], mask)
        hp = jnp.dot(p, w, preferred_element_type=jnp.float32)
        return _elu(hp)

    h1 = jnp.concatenate(
        [head(wa_ref, a1a_ref), head(wb_ref, a1b_ref)], axis=1)
    who_ref[...] = jnp.dot(h1, wout_ref[...],
                           preferred_element_type=jnp.float32)


def _layer2_body(adj_ref, who_ref, aout_ref, out_ref):
    i = pl.program_id(0)
    mask = adj_ref[...] > 0.0
    w = who_ref[...]
    a_row = aout_ref[0, :]
    wrows = who_ref[pl.ds(i * R, R), :]
    f1 = jnp.sum(wrows * a_row[:NCLASS][None, :], axis=1)
    f2 = jnp.sum(w * a_row[NCLASS:][None, :], axis=1)
    p = _masked_softmax(f1[:, None] + f2[None, :], mask)
    hp = jnp.dot(p, w, preferred_element_type=jnp.float32)
    out = _elu(hp)
    m = jnp.max(out, axis=1, keepdims=True)
    lse = jnp.log(jnp.sum(jnp.exp(out - m), axis=1, keepdims=True))
    out_ref[...] = out - m - lse


def _full(shape):
    return pl.BlockSpec(shape, lambda i: (0, 0))


def kernel(idx, adj, table, W1a, a1a, W1b, a1b, Wout, aout):
    features = _sc_gather(table, idx)

    wa, wb = pl.pallas_call(
        _prewh_body,
        grid=(N // R,),
        in_specs=[pl.BlockSpec((R, D), lambda i: (i, 0)),
                  _full((D, NHID)), _full((D, NHID))],
        out_specs=[pl.BlockSpec((R, NHID), lambda i: (i, 0)),
                   pl.BlockSpec((R, NHID), lambda i: (i, 0))],
        out_shape=[jax.ShapeDtypeStruct((N, NHID), jnp.float32),
                   jax.ShapeDtypeStruct((N, NHID), jnp.float32)],
    )(features, W1a, W1b)

    who = pl.pallas_call(
        _layer1_body,
        grid=(N // R,),
        in_specs=[pl.BlockSpec((R, N), lambda i: (i, 0)),
                  _full((N, NHID)), _full((N, NHID)),
                  _full((1, 2 * NHID)), _full((1, 2 * NHID)),
                  _full((2 * NHID, NCLASS))],
        out_specs=pl.BlockSpec((R, NCLASS), lambda i: (i, 0)),
        out_shape=jax.ShapeDtypeStruct((N, NCLASS), jnp.float32),
    )(adj, wa, wb, a1a.reshape(1, -1), a1b.reshape(1, -1), Wout)

    out = pl.pallas_call(
        _layer2_body,
        grid=(N // R,),
        in_specs=[pl.BlockSpec((R, N), lambda i: (i, 0)),
                  _full((N, NCLASS)), _full((1, 2 * NCLASS))],
        out_specs=pl.BlockSpec((R, NCLASS), lambda i: (i, 0)),
        out_shape=jax.ShapeDtypeStruct((N, NCLASS), jnp.float32),
    )(adj, who, aout.reshape(1, -1))

    return out


# speedup vs baseline: 1.4440x; 1.4440x over previous
import jax
import jax.numpy as jnp
from jax.experimental import pallas as pl
from jax.experimental.pallas import tpu as pltpu
from jax.experimental.pallas import tpu_sc as plsc

N = 4096
D = 128
NHID = 128
NCLASS = 128
ALPHA = 0.2
R = 256
GW = 128
NEG = -9e15


def _sc_gather(table, idx):
    n = idx.shape[0]
    d = table.shape[1]
    idx2 = idx.reshape(1, n)
    mesh = plsc.VectorSubcoreMesh(core_axis_name="core",
                                  subcore_axis_name="subcore")

    @pl.kernel(out_type=jax.ShapeDtypeStruct((n, d), table.dtype), mesh=mesh)
    def gather_kernel(tab_hbm, i_hbm, o_hbm):
        def body(i_vmem, o_vmem):
            pltpu.sync_copy(tab_hbm.at[i_vmem.at[0]], o_vmem)

        pltpu.emit_pipeline(
            body,
            grid=(n // GW,),
            in_specs=[pl.BlockSpec((1, GW), index_map=lambda i: (0, i))],
            out_specs=[pl.BlockSpec((GW, d), index_map=lambda i: (i, 0))],
            core_axis_name=("core", "subcore"),
            dimension_semantics=(pltpu.PARALLEL,),
        )(i_hbm, o_hbm)

    return gather_kernel(table, idx2)


def _prewh_body(x_ref, w1a_ref, w1b_ref, wa_ref, wb_ref):
    x = x_ref[...]
    wa_ref[...] = jnp.dot(x, w1a_ref[...], preferred_element_type=jnp.float32)
    wb_ref[...] = jnp.dot(x, w1b_ref[...], preferred_element_type=jnp.float32)


def _masked_softmax(e, mask):
    e = jnp.where(e >= 0, e, ALPHA * e)
    e = jnp.where(mask, e, NEG)
    m = jnp.max(e, axis=1, keepdims=True)
    p = jnp.exp(e - m)
    return p / jnp.sum(p, axis=1, keepdims=True)


def _elu(x):
    return jnp.where(x > 0, x, jnp.exp(x) - 1.0)


def _layer1_body(adj_ref, wa_ref, wb_ref, a1a_ref, a1b_ref, wout_ref,
                 who_ref):
    i = pl.program_id(0)
    mask = adj_ref[...] > 0.0

    def head(w_ref, a_ref):
        w = w_ref[...]
        a_row = a_ref[0, :]
        wrows = w_ref[pl.ds(i * R, R), :]
        f1 = jnp.sum(wrows * a_row[:D][None, :], axis=1)
        f2 = jnp.sum(w * a_row[D:][None, :], axis=1)
        p = _masked_softmax(f1[:, None] + f2[None, :], mask)
        hp = jnp.dot(p, w, preferred_element_type=jnp.float32)
        return _elu(hp)

    h1 = jnp.concatenate(
        [head(wa_ref, a1a_ref), head(wb_ref, a1b_ref)], axis=1)
    who_ref[...] = jnp.dot(h1, wout_ref[...],
                           preferred_element_type=jnp.float32)


def _layer2_body(adj_ref, who_ref, aout_ref, out_ref):
    i = pl.program_id(0)
    mask = adj_ref[...] > 0.0
    w = who_ref[...]
    a_row = aout_ref[0, :]
    wrows = who_ref[pl.ds(i * R, R), :]
    f1 = jnp.sum(wrows * a_row[:NCLASS][None, :], axis=1)
    f2 = jnp.sum(w * a_row[NCLASS:][None, :], axis=1)
    p = _masked_softmax(f1[:, None] + f2[None, :], mask)
    hp = jnp.dot(p, w, preferred_element_type=jnp.float32)
    out = _elu(hp)
    m = jnp.max(out, axis=1, keepdims=True)
    lse = jnp.log(jnp.sum(jnp.exp(out - m), axis=1, keepdims=True))
    out_ref[...] = out - m - lse


def _full(shape):
    return pl.BlockSpec(shape, lambda i: (0, 0))


def kernel(idx, adj, table, W1a, a1a, W1b, a1b, Wout, aout):
    features = _sc_gather(table, idx)

    wa, wb = pl.pallas_call(
        _prewh_body,
        grid=(N // R,),
        in_specs=[pl.BlockSpec((R, D), lambda i: (i, 0)),
                  _full((D, NHID)), _full((D, NHID))],
        out_specs=[pl.BlockSpec((R, NHID), lambda i: (i, 0)),
                   pl.BlockSpec((R, NHID), lambda i: (i, 0))],
        out_shape=[jax.ShapeDtypeStruct((N, NHID), jnp.float32),
                   jax.ShapeDtypeStruct((N, NHID), jnp.float32)],
    )(features, W1a, W1b)

    who = pl.pallas_call(
        _layer1_body,
        grid=(N // R,),
        in_specs=[pl.BlockSpec((R, N), lambda i: (i, 0)),
                  _full((N, NHID)), _full((N, NHID)),
                  _full((1, 2 * NHID)), _full((1, 2 * NHID)),
                  _full((2 * NHID, NCLASS))],
        out_specs=pl.BlockSpec((R, NCLASS), lambda i: (i, 0)),
        out_shape=jax.ShapeDtypeStruct((N, NCLASS), jnp.float32),
    )(adj, wa, wb, a1a.reshape(1, -1), a1b.reshape(1, -1), Wout)

    out = pl.pallas_call(
        _layer2_body,
        grid=(N // R,),
        in_specs=[pl.BlockSpec((R, N), lambda i: (i, 0)),
                  _full((N, NCLASS)), _full((1, 2 * NCLASS))],
        out_specs=pl.BlockSpec((R, NCLASS), lambda i: (i, 0)),
        out_shape=jax.ShapeDtypeStruct((N, NCLASS), jnp.float32),
    )(adj, who, aout.reshape(1, -1))

    return out


# speedup vs baseline: 1.6416x; 1.1369x over previous
import jax
import jax.numpy as jnp
from jax.experimental import pallas as pl
from jax.experimental.pallas import tpu as pltpu
from jax.experimental.pallas import tpu_sc as plsc

N = 4096
D = 128
NHID = 128
NCLASS = 128
ALPHA = 0.2
R = 256
GW = 128
NEG = -9e15


def _sc_gather(table, idx):
    n = idx.shape[0]
    d = table.shape[1]
    idx2 = idx.reshape(1, n)
    mesh = plsc.VectorSubcoreMesh(core_axis_name="core",
                                  subcore_axis_name="subcore")

    @pl.kernel(out_type=jax.ShapeDtypeStruct((n, d), table.dtype), mesh=mesh)
    def gather_kernel(tab_hbm, i_hbm, o_hbm):
        def body(i_vmem, o_vmem):
            pltpu.sync_copy(tab_hbm.at[i_vmem.at[0]], o_vmem)

        pltpu.emit_pipeline(
            body,
            grid=(n // GW,),
            in_specs=[pl.BlockSpec((1, GW), index_map=lambda i: (0, i))],
            out_specs=[pl.BlockSpec((GW, d), index_map=lambda i: (i, 0))],
            core_axis_name=("core", "subcore"),
            dimension_semantics=(pltpu.PARALLEL,),
        )(i_hbm, o_hbm)

    return gather_kernel(table, idx2)


def _logit_vecs(wh, a_row):
    a1 = a_row[:, :D]
    a2 = a_row[:, D:]
    f1 = jnp.sum(wh * a1, axis=1, keepdims=True)
    f2 = jax.lax.dot_general(a2, wh, (((1,), (1,)), ((), ())),
                             preferred_element_type=jnp.float32)
    return f1, f2


def _prewh_body(x_ref, w1a_ref, w1b_ref, a1a_ref, a1b_ref,
                wa_ref, wb_ref, f1a_ref, f2a_ref, f1b_ref, f2b_ref):
    x = x_ref[...]
    wa = jnp.dot(x, w1a_ref[...], preferred_element_type=jnp.float32)
    wb = jnp.dot(x, w1b_ref[...], preferred_element_type=jnp.float32)
    wa_ref[...] = wa
    wb_ref[...] = wb
    f1a_ref[...], f2a_ref[...] = _logit_vecs(wa, a1a_ref[...])
    f1b_ref[...], f2b_ref[...] = _logit_vecs(wb, a1b_ref[...])


def _unnorm_att(f1_col, f2_row, mask):
    e = f1_col + f2_row
    e = jnp.maximum(e, ALPHA * e)
    e = jnp.where(mask, e, NEG)
    m = jnp.max(e, axis=1, keepdims=True)
    p = jnp.exp(e - m)
    s = jnp.sum(p, axis=1, keepdims=True)
    return p, s


def _elu(x):
    return jnp.where(x > 0, x, jnp.exp(x) - 1.0)


def _layer1_body(adj_ref, wa_ref, wb_ref, f1a_ref, f2a_ref, f1b_ref,
                 f2b_ref, wout_ref, aout_ref,
                 who_ref, f1o_ref, f2o_ref):
    mask = adj_ref[...] > 0.0

    def head(w_ref, f1_ref, f2_ref):
        p, s = _unnorm_att(f1_ref[...], f2_ref[...], mask)
        hp = jnp.dot(p, w_ref[...], preferred_element_type=jnp.float32)
        return _elu(hp / s)

    h1 = jnp.concatenate(
        [head(wa_ref, f1a_ref, f2a_ref), head(wb_ref, f1b_ref, f2b_ref)],
        axis=1)
    who = jnp.dot(h1, wout_ref[...], preferred_element_type=jnp.float32)
    who_ref[...] = who
    f1o_ref[...], f2o_ref[...] = _logit_vecs(who, aout_ref[...])


def _layer2_body(adj_ref, who_ref, f1o_ref, f2o_ref, out_ref):
    mask = adj_ref[...] > 0.0
    p, s = _unnorm_att(f1o_ref[...], f2o_ref[...], mask)
    hp = jnp.dot(p, who_ref[...], preferred_element_type=jnp.float32)
    out = _elu(hp / s)
    m = jnp.max(out, axis=1, keepdims=True)
    lse = jnp.log(jnp.sum(jnp.exp(out - m), axis=1, keepdims=True))
    out_ref[...] = out - m - lse


def _full(shape):
    return pl.BlockSpec(shape, lambda i: (0, 0))


def _rowblk(cols):
    return pl.BlockSpec((R, cols), lambda i: (i, 0))


def _colvec():
    return pl.BlockSpec((R, 1), lambda i: (i, 0))


def _rowvec():
    return pl.BlockSpec((1, R), lambda i: (0, i))


def kernel(idx, adj, table, W1a, a1a, W1b, a1b, Wout, aout):
    features = _sc_gather(table, idx)

    f32 = jnp.float32
    wa, wb, f1a, f2a, f1b, f2b = pl.pallas_call(
        _prewh_body,
        grid=(N // R,),
        in_specs=[_rowblk(D), _full((D, NHID)), _full((D, NHID)),
                  _full((1, 2 * NHID)), _full((1, 2 * NHID))],
        out_specs=[_rowblk(NHID), _rowblk(NHID),
                   _colvec(), _rowvec(), _colvec(), _rowvec()],
        out_shape=[jax.ShapeDtypeStruct((N, NHID), f32),
                   jax.ShapeDtypeStruct((N, NHID), f32),
                   jax.ShapeDtypeStruct((N, 1), f32),
                   jax.ShapeDtypeStruct((1, N), f32),
                   jax.ShapeDtypeStruct((N, 1), f32),
                   jax.ShapeDtypeStruct((1, N), f32)],
    )(features, W1a, W1b, a1a.reshape(1, -1), a1b.reshape(1, -1))

    who, f1o, f2o = pl.pallas_call(
        _layer1_body,
        grid=(N // R,),
        in_specs=[_rowblk(N), _full((N, NHID)), _full((N, NHID)),
                  _colvec(), _full((1, N)), _colvec(), _full((1, N)),
                  _full((2 * NHID, NCLASS)), _full((1, 2 * NCLASS))],
        out_specs=[_rowblk(NCLASS), _colvec(), _rowvec()],
        out_shape=[jax.ShapeDtypeStruct((N, NCLASS), f32),
                   jax.ShapeDtypeStruct((N, 1), f32),
                   jax.ShapeDtypeStruct((1, N), f32)],
    )(adj, wa, wb, f1a, f2a, f1b, f2b, Wout, aout.reshape(1, -1))

    out = pl.pallas_call(
        _layer2_body,
        grid=(N // R,),
        in_specs=[_rowblk(N), _full((N, NCLASS)), _colvec(), _full((1, N))],
        out_specs=_rowblk(NCLASS),
        out_shape=jax.ShapeDtypeStruct((N, NCLASS), f32),
    )(adj, who, f1o, f2o)

    return out


# speedup vs baseline: 1.7780x; 1.0830x over previous
import jax
import jax.numpy as jnp
from jax.experimental import pallas as pl
from jax.experimental.pallas import tpu as pltpu
from jax.experimental.pallas import tpu_sc as plsc

N = 4096
D = 128
NHID = 128
NCLASS = 128
ALPHA = 0.2
R = 256
GW = 128


def _sc_gather(table, idx):
    n = idx.shape[0]
    d = table.shape[1]
    idx2 = idx.reshape(1, n)
    mesh = plsc.VectorSubcoreMesh(core_axis_name="core",
                                  subcore_axis_name="subcore")

    @pl.kernel(out_type=jax.ShapeDtypeStruct((n, d), table.dtype), mesh=mesh)
    def gather_kernel(tab_hbm, i_hbm, o_hbm):
        def body(i_vmem, o_vmem):
            pltpu.sync_copy(tab_hbm.at[i_vmem.at[0]], o_vmem)

        pltpu.emit_pipeline(
            body,
            grid=(n // GW,),
            in_specs=[pl.BlockSpec((1, GW), index_map=lambda i: (0, i))],
            out_specs=[pl.BlockSpec((GW, d), index_map=lambda i: (i, 0))],
            core_axis_name=("core", "subcore"),
            dimension_semantics=(pltpu.PARALLEL,),
        )(i_hbm, o_hbm)

    return gather_kernel(table, idx2)


def _leaky(x):
    return jnp.maximum(x, ALPHA * x)


def _elu(x):
    return jnp.where(x > 0, x, jnp.exp(x) - 1.0)


def _logit_vecs(wh, a_row):
    a1 = a_row[:, :D]
    a2 = a_row[:, D:]
    f1 = jnp.sum(wh * a1, axis=1, keepdims=True)
    f2 = jax.lax.dot_general(a2, wh, (((1,), (1,)), ((), ())),
                             preferred_element_type=jnp.float32)
    return f1, f2


def _prewh_body(x_ref, w1a_ref, w1b_ref, a1a_ref, a1b_ref,
                wa_ref, wb_ref, f1a_ref, f2a_ref, f1b_ref, f2b_ref,
                cma_ref, cmb_ref):
    i = pl.program_id(0)

    @pl.when(i == 0)
    def _():
        cma_ref[...] = jnp.zeros_like(cma_ref)
        cmb_ref[...] = jnp.zeros_like(cmb_ref)

    x = x_ref[...]
    wa = jnp.dot(x, w1a_ref[...], preferred_element_type=jnp.float32)
    wb = jnp.dot(x, w1b_ref[...], preferred_element_type=jnp.float32)
    wa_ref[...] = wa
    wb_ref[...] = wb
    f1a_ref[...], f2a_ref[...] = _logit_vecs(wa, a1a_ref[...])
    f1b_ref[...], f2b_ref[...] = _logit_vecs(wb, a1b_ref[...])
    cma_ref[...] += jnp.sum(wa, axis=0, keepdims=True) * (1.0 / N)
    cmb_ref[...] += jnp.sum(wb, axis=0, keepdims=True) * (1.0 / N)


def _att_block(adjv, f1, f2, w, cm):
    m = _leaky(f1 + jnp.max(f2))
    e = f1 + f2
    p = adjv * jnp.exp(_leaky(e) - m)
    s = jnp.sum(p, axis=1, keepdims=True)
    hp = jnp.dot(p, w, preferred_element_type=jnp.float32)
    safe = jnp.where(s > 0, s, 1.0)
    return jnp.where(s > 0, hp / safe, cm)


def _layer1_body(adj_ref, wa_ref, wb_ref, f1a_ref, f2a_ref, f1b_ref,
                 f2b_ref, cma_ref, cmb_ref, wout_ref, aout_ref,
                 who_ref, f1o_ref, f2o_ref, cmo_ref):
    i = pl.program_id(0)

    @pl.when(i == 0)
    def _():
        cmo_ref[...] = jnp.zeros_like(cmo_ref)

    adjv = adj_ref[...]
    ha = _elu(_att_block(adjv, f1a_ref[...], f2a_ref[...], wa_ref[...],
                         cma_ref[...]))
    hb = _elu(_att_block(adjv, f1b_ref[...], f2b_ref[...], wb_ref[...],
                         cmb_ref[...]))
    h1 = jnp.concatenate([ha, hb], axis=1)
    who = jnp.dot(h1, wout_ref[...], preferred_element_type=jnp.float32)
    who_ref[...] = who
    f1o_ref[...], f2o_ref[...] = _logit_vecs(who, aout_ref[...])
    cmo_ref[...] += jnp.sum(who, axis=0, keepdims=True) * (1.0 / N)


def _layer2_body(adj_ref, who_ref, f1o_ref, f2o_ref, cmo_ref, out_ref):
    out = _elu(_att_block(adj_ref[...], f1o_ref[...], f2o_ref[...],
                          who_ref[...], cmo_ref[...]))
    m = jnp.max(out, axis=1, keepdims=True)
    lse = jnp.log(jnp.sum(jnp.exp(out - m), axis=1, keepdims=True))
    out_ref[...] = out - m - lse


def _full(shape):
    return pl.BlockSpec(shape, lambda i: (0, 0))


def _rowblk(cols):
    return pl.BlockSpec((R, cols), lambda i: (i, 0))


def _colvec():
    return pl.BlockSpec((R, 1), lambda i: (i, 0))


def _rowvec():
    return pl.BlockSpec((1, R), lambda i: (0, i))


def kernel(idx, adj, table, W1a, a1a, W1b, a1b, Wout, aout):
    features = _sc_gather(table, idx)

    f32 = jnp.float32
    wa, wb, f1a, f2a, f1b, f2b, cma, cmb = pl.pallas_call(
        _prewh_body,
        grid=(N // R,),
        in_specs=[_rowblk(D), _full((D, NHID)), _full((D, NHID)),
                  _full((1, 2 * NHID)), _full((1, 2 * NHID))],
        out_specs=[_rowblk(NHID), _rowblk(NHID),
                   _colvec(), _rowvec(), _colvec(), _rowvec(),
                   _full((1, NHID)), _full((1, NHID))],
        out_shape=[jax.ShapeDtypeStruct((N, NHID), f32),
                   jax.ShapeDtypeStruct((N, NHID), f32),
                   jax.ShapeDtypeStruct((N, 1), f32),
                   jax.ShapeDtypeStruct((1, N), f32),
                   jax.ShapeDtypeStruct((N, 1), f32),
                   jax.ShapeDtypeStruct((1, N), f32),
                   jax.ShapeDtypeStruct((1, NHID), f32),
                   jax.ShapeDtypeStruct((1, NHID), f32)],
    )(features, W1a, W1b, a1a.reshape(1, -1), a1b.reshape(1, -1))

    who, f1o, f2o, cmo = pl.pallas_call(
        _layer1_body,
        grid=(N // R,),
        in_specs=[_rowblk(N), _full((N, NHID)), _full((N, NHID)),
                  _colvec(), _full((1, N)), _colvec(), _full((1, N)),
                  _full((1, NHID)), _full((1, NHID)),
                  _full((2 * NHID, NCLASS)), _full((1, 2 * NCLASS))],
        out_specs=[_rowblk(NCLASS), _colvec(), _rowvec(),
                   _full((1, NCLASS))],
        out_shape=[jax.ShapeDtypeStruct((N, NCLASS), f32),
                   jax.ShapeDtypeStruct((N, 1), f32),
                   jax.ShapeDtypeStruct((1, N), f32),
                   jax.ShapeDtypeStruct((1, NCLASS), f32)],
    )(adj, wa, wb, f1a, f2a, f1b, f2b, cma, cmb, Wout,
      aout.reshape(1, -1))

    out = pl.pallas_call(
        _layer2_body,
        grid=(N // R,),
        in_specs=[_rowblk(N), _full((N, NCLASS)), _colvec(), _full((1, N)),
                  _full((1, NCLASS))],
        out_specs=_rowblk(NCLASS),
        out_shape=jax.ShapeDtypeStruct((N, NCLASS), f32),
    )(adj, who, f1o, f2o, cmo)

    return out


# speedup vs baseline: 1.9102x; 1.0744x over previous
import jax
import jax.numpy as jnp
from jax.experimental import pallas as pl
from jax.experimental.pallas import tpu as pltpu
from jax.experimental.pallas import tpu_sc as plsc

N = 4096
D = 128
NHID = 128
NCLASS = 128
ALPHA = 0.2
R = 256
GW = 128


def _sc_gather(table, idx):
    n = idx.shape[0]
    d = table.shape[1]
    idx2 = idx.reshape(1, n)
    mesh = plsc.VectorSubcoreMesh(core_axis_name="core",
                                  subcore_axis_name="subcore")

    @pl.kernel(out_type=jax.ShapeDtypeStruct((n, d), table.dtype), mesh=mesh)
    def gather_kernel(tab_hbm, i_hbm, o_hbm):
        def body(i_vmem, o_vmem):
            pltpu.sync_copy(tab_hbm.at[i_vmem.at[0]], o_vmem)

        pltpu.emit_pipeline(
            body,
            grid=(n // GW,),
            in_specs=[pl.BlockSpec((1, GW), index_map=lambda i: (0, i))],
            out_specs=[pl.BlockSpec((GW, d), index_map=lambda i: (i, 0))],
            core_axis_name=("core", "subcore"),
            dimension_semantics=(pltpu.PARALLEL,),
        )(i_hbm, o_hbm)

    return gather_kernel(table, idx2)


def _leaky(x):
    return jnp.maximum(x, ALPHA * x)


def _elu(x):
    return jnp.where(x > 0, x, jnp.exp(x) - 1.0)


LOG2E = 1.4426950408889634


def _logit_vecs(wh, a_row):
    a1 = a_row[:, :D] * LOG2E
    a2 = a_row[:, D:] * LOG2E
    f1 = jnp.sum(wh * a1, axis=1, keepdims=True)
    f2 = jax.lax.dot_general(a2, wh, (((1,), (1,)), ((), ())),
                             preferred_element_type=jnp.float32)
    return f1, f2


def _prewh_body(x_ref, w1a_ref, w1b_ref, a1a_ref, a1b_ref,
                wa_ref, wb_ref, f1a_ref, f2a_ref, f1b_ref, f2b_ref,
                cma_ref, cmb_ref):
    i = pl.program_id(0)

    @pl.when(i == 0)
    def _():
        cma_ref[...] = jnp.zeros_like(cma_ref)
        cmb_ref[...] = jnp.zeros_like(cmb_ref)

    x = x_ref[...]
    wa = jnp.dot(x, w1a_ref[...], preferred_element_type=jnp.float32)
    wb = jnp.dot(x, w1b_ref[...], preferred_element_type=jnp.float32)
    wa_ref[...] = wa
    wb_ref[...] = wb
    f1a_ref[...], f2a_ref[...] = _logit_vecs(wa, a1a_ref[...])
    f1b_ref[...], f2b_ref[...] = _logit_vecs(wb, a1b_ref[...])
    cma_ref[...] += jnp.sum(wa, axis=0, keepdims=True) * (1.0 / N)
    cmb_ref[...] += jnp.sum(wb, axis=0, keepdims=True) * (1.0 / N)


def _att_block(adjv, f1, f2, w, cm):
    m = _leaky(f1 + jnp.max(f2))
    u_row = f1 - m
    v_row = ALPHA * f1 - m
    f2s = ALPHA * f2
    x = jnp.maximum(u_row + f2, v_row + f2s)
    p = adjv * jnp.exp2(x)
    s = jnp.sum(p, axis=1, keepdims=True)
    hp = jnp.dot(p, w, preferred_element_type=jnp.float32)
    safe = jnp.where(s > 0, s, 1.0)
    return jnp.where(s > 0, hp / safe, cm)


def _layer1_body(adj_ref, wa_ref, wb_ref, f1a_ref, f2a_ref, f1b_ref,
                 f2b_ref, cma_ref, cmb_ref, wout_ref, aout_ref,
                 who_ref, f1o_ref, f2o_ref, cmo_ref):
    i = pl.program_id(0)

    @pl.when(i == 0)
    def _():
        cmo_ref[...] = jnp.zeros_like(cmo_ref)

    adjv = adj_ref[...]
    ha = _elu(_att_block(adjv, f1a_ref[...], f2a_ref[...], wa_ref[...],
                         cma_ref[...]))
    hb = _elu(_att_block(adjv, f1b_ref[...], f2b_ref[...], wb_ref[...],
                         cmb_ref[...]))
    h1 = jnp.concatenate([ha, hb], axis=1)
    who = jnp.dot(h1, wout_ref[...], preferred_element_type=jnp.float32)
    who_ref[...] = who
    f1o_ref[...], f2o_ref[...] = _logit_vecs(who, aout_ref[...])
    cmo_ref[...] += jnp.sum(who, axis=0, keepdims=True) * (1.0 / N)


def _layer2_body(adj_ref, who_ref, f1o_ref, f2o_ref, cmo_ref, out_ref):
    out = _elu(_att_block(adj_ref[...], f1o_ref[...], f2o_ref[...],
                          who_ref[...], cmo_ref[...]))
    m = jnp.max(out, axis=1, keepdims=True)
    lse = jnp.log(jnp.sum(jnp.exp(out - m), axis=1, keepdims=True))
    out_ref[...] = out - m - lse


def _full(shape):
    return pl.BlockSpec(shape, lambda i: (0, 0))


def _rowblk(cols):
    return pl.BlockSpec((R, cols), lambda i: (i, 0))


def _colvec():
    return pl.BlockSpec((R, 1), lambda i: (i, 0))


def _rowvec():
    return pl.BlockSpec((1, R), lambda i: (0, i))


def kernel(idx, adj, table, W1a, a1a, W1b, a1b, Wout, aout):
    features = _sc_gather(table, idx)

    f32 = jnp.float32
    wa, wb, f1a, f2a, f1b, f2b, cma, cmb = pl.pallas_call(
        _prewh_body,
        grid=(N // R,),
        in_specs=[_rowblk(D), _full((D, NHID)), _full((D, NHID)),
                  _full((1, 2 * NHID)), _full((1, 2 * NHID))],
        out_specs=[_rowblk(NHID), _rowblk(NHID),
                   _colvec(), _rowvec(), _colvec(), _rowvec(),
                   _full((1, NHID)), _full((1, NHID))],
        out_shape=[jax.ShapeDtypeStruct((N, NHID), f32),
                   jax.ShapeDtypeStruct((N, NHID), f32),
                   jax.ShapeDtypeStruct((N, 1), f32),
                   jax.ShapeDtypeStruct((1, N), f32),
                   jax.ShapeDtypeStruct((N, 1), f32),
                   jax.ShapeDtypeStruct((1, N), f32),
                   jax.ShapeDtypeStruct((1, NHID), f32),
                   jax.ShapeDtypeStruct((1, NHID), f32)],
    )(features, W1a, W1b, a1a.reshape(1, -1), a1b.reshape(1, -1))

    who, f1o, f2o, cmo = pl.pallas_call(
        _layer1_body,
        grid=(N // R,),
        in_specs=[_rowblk(N), _full((N, NHID)), _full((N, NHID)),
                  _colvec(), _full((1, N)), _colvec(), _full((1, N)),
                  _full((1, NHID)), _full((1, NHID)),
                  _full((2 * NHID, NCLASS)), _full((1, 2 * NCLASS))],
        out_specs=[_rowblk(NCLASS), _colvec(), _rowvec(),
                   _full((1, NCLASS))],
        out_shape=[jax.ShapeDtypeStruct((N, NCLASS), f32),
                   jax.ShapeDtypeStruct((N, 1), f32),
                   jax.ShapeDtypeStruct((1, N), f32),
                   jax.ShapeDtypeStruct((1, NCLASS), f32)],
    )(adj, wa, wb, f1a, f2a, f1b, f2b, cma, cmb, Wout,
      aout.reshape(1, -1))

    out = pl.pallas_call(
        _layer2_body,
        grid=(N // R,),
        in_specs=[_rowblk(N), _full((N, NCLASS)), _colvec(), _full((1, N)),
                  _full((1, NCLASS))],
        out_specs=_rowblk(NCLASS),
        out_shape=jax.ShapeDtypeStruct((N, NCLASS), f32),
    )(adj, who, f1o, f2o, cmo)

    return out


# speedup vs baseline: 2.1881x; 1.1455x over previous
import jax
import jax.numpy as jnp
from jax.experimental import pallas as pl
from jax.experimental.pallas import tpu as pltpu
from jax.experimental.pallas import tpu_sc as plsc

N = 4096
D = 128
NHID = 128
NCLASS = 128
ALPHA = 0.2
R = 256
GW = 128
NBLK = N // R
LOG2E = 1.4426950408889634


def _sc_gather(table, idx):
    n = idx.shape[0]
    d = table.shape[1]
    idx2 = idx.reshape(1, n)
    mesh = plsc.VectorSubcoreMesh(core_axis_name="core",
                                  subcore_axis_name="subcore")

    @pl.kernel(out_type=jax.ShapeDtypeStruct((n, d), table.dtype), mesh=mesh)
    def gather_kernel(tab_hbm, i_hbm, o_hbm):
        def body(i_vmem, o_vmem):
            pltpu.sync_copy(tab_hbm.at[i_vmem.at[0]], o_vmem)

        pltpu.emit_pipeline(
            body,
            grid=(n // GW,),
            in_specs=[pl.BlockSpec((1, GW), index_map=lambda i: (0, i))],
            out_specs=[pl.BlockSpec((GW, d), index_map=lambda i: (i, 0))],
            core_axis_name=("core", "subcore"),
            dimension_semantics=(pltpu.PARALLEL,),
        )(i_hbm, o_hbm)

    return gather_kernel(table, idx2)


def _leaky(x):
    return jnp.maximum(x, ALPHA * x)


def _elu(x):
    return jnp.where(x > 0, x, jnp.exp(x) - 1.0)


def _logit_vecs(wh, a_row):
    a1 = a_row[:, :D] * LOG2E
    a2 = a_row[:, D:] * LOG2E
    f1 = jnp.sum(wh * a1, axis=1, keepdims=True)
    f2 = jax.lax.dot_general(a2, wh, (((1,), (1,)), ((), ())),
                             preferred_element_type=jnp.float32)
    return f1, f2


def _att_block(adjv, f1, f2, w, cm):
    m = _leaky(f1 + jnp.max(f2))
    u_row = f1 - m
    v_row = ALPHA * f1 - m
    f2s = ALPHA * f2
    x = jnp.maximum(u_row + f2, v_row + f2s)
    p = adjv * jnp.exp2(x)
    s = jnp.sum(p, axis=1, keepdims=True)
    hp = jnp.dot(p, w, preferred_element_type=jnp.float32)
    safe = jnp.where(s > 0, s, 1.0)
    return jnp.where(s > 0, hp / safe, cm)


def _mega_body(x_ref, adj_ref, w1a_ref, w1b_ref, a1a_ref, a1b_ref,
               wout_ref, aout_ref, out_ref,
               wa_ref, wb_ref, f1a_ref, f2a_ref, f1b_ref, f2b_ref,
               cma_ref, cmb_ref,
               who_ref, f1o_ref, f2o_ref, cmo_ref):
    i = pl.program_id(0)

    @pl.when(i == 0)
    def _prewh():
        x = x_ref[...]
        wa = jnp.dot(x, w1a_ref[...], preferred_element_type=jnp.float32)
        wb = jnp.dot(x, w1b_ref[...], preferred_element_type=jnp.float32)
        wa_ref[...] = wa
        wb_ref[...] = wb
        f1a_ref[...], f2a_ref[...] = _logit_vecs(wa, a1a_ref[...])
        f1b_ref[...], f2b_ref[...] = _logit_vecs(wb, a1b_ref[...])
        cma_ref[...] = jnp.mean(wa, axis=0, keepdims=True)
        cmb_ref[...] = jnp.mean(wb, axis=0, keepdims=True)
        cmo_ref[...] = jnp.zeros_like(cmo_ref)

    @pl.when((i >= 1) & (i <= NBLK))
    def _layer1():
        r0 = (i - 1) * R
        adjv = adj_ref[...]
        ha = _elu(_att_block(adjv, f1a_ref[pl.ds(r0, R), :], f2a_ref[...],
                             wa_ref[...], cma_ref[...]))
        hb = _elu(_att_block(adjv, f1b_ref[pl.ds(r0, R), :], f2b_ref[...],
                             wb_ref[...], cmb_ref[...]))
        h1 = jnp.concatenate([ha, hb], axis=1)
        who = jnp.dot(h1, wout_ref[...], preferred_element_type=jnp.float32)
        who_ref[pl.ds(r0, R), :] = who
        f1o, f2o = _logit_vecs(who, aout_ref[...])
        f1o_ref[pl.ds(r0, R), :] = f1o
        f2o_ref[:, pl.ds(r0, R)] = f2o
        cmo_ref[...] += jnp.sum(who, axis=0, keepdims=True) * (1.0 / N)

    @pl.when(i > NBLK)
    def _layer2():
        r0 = (i - NBLK - 1) * R
        out = _elu(_att_block(adj_ref[...], f1o_ref[pl.ds(r0, R), :],
                              f2o_ref[...], who_ref[...], cmo_ref[...]))
        m = jnp.max(out, axis=1, keepdims=True)
        lse = jnp.log(jnp.sum(jnp.exp(out - m), axis=1, keepdims=True))
        out_ref[...] = out - m - lse


def _full(shape):
    return pl.BlockSpec(shape, lambda i: (0, 0))


def kernel(idx, adj, table, W1a, a1a, W1b, a1b, Wout, aout):
    features = _sc_gather(table, idx)

    f32 = jnp.float32
    vmem = pltpu.VMEM
    out = pl.pallas_call(
        _mega_body,
        grid=(2 * NBLK + 1,),
        in_specs=[_full((N, D)),
                  pl.BlockSpec((R, N), lambda i: ((i + NBLK - 1) % NBLK, 0)),
                  _full((D, NHID)), _full((D, NHID)),
                  _full((1, 2 * NHID)), _full((1, 2 * NHID)),
                  _full((2 * NHID, NCLASS)), _full((1, 2 * NCLASS))],
        out_specs=pl.BlockSpec(
            (R, NCLASS), lambda i: (jnp.maximum(i - NBLK - 1, 0), 0)),
        out_shape=jax.ShapeDtypeStruct((N, NCLASS), f32),
        scratch_shapes=[
            vmem((N, NHID), f32), vmem((N, NHID), f32),
            vmem((N, 1), f32), vmem((1, N), f32),
            vmem((N, 1), f32), vmem((1, N), f32),
            vmem((1, NHID), f32), vmem((1, NHID), f32),
            vmem((N, NCLASS), f32),
            vmem((N, 1), f32), vmem((1, N), f32),
            vmem((1, NCLASS), f32),
        ],
    )(features, adj, W1a, W1b, a1a.reshape(1, -1), a1b.reshape(1, -1),
      Wout, aout.reshape(1, -1))

    return out
